# baseline (speedup 1.0000x reference)
import jax
import jax.numpy as jnp
from jax import lax
from jax.experimental import pallas as pl
from jax.experimental.pallas import tpu as pltpu
from jax.experimental.pallas import tpu_sc as plsc

N_FIELDS = 26
VOCAB = 100000
DIM = 16
BATCH = 16384

NC = 2
NS = 16
NW = NC * NS

TOTAL_ROWS = BATCH * N_FIELDS
GRP = 128
N_GRPS = TOTAL_ROWS // GRP
GRPS_PER_W = N_GRPS // NW
GRPS_PER_CHUNK = 8
CHUNK_ROWS = GRPS_PER_CHUNK * GRP
CHUNKS_PER_W = GRPS_PER_W // GRPS_PER_CHUNK


def _gather_kernel(idx_hbm, table_hbm, out_hbm, idx_v, rows_v, sem):
    wid = lax.axis_index("s") * NC + lax.axis_index("c")
    grp0 = wid * GRPS_PER_W

    @pl.loop(0, CHUNKS_PER_W)
    def _chunk(i):
        g = grp0 + i * GRPS_PER_CHUNK
        pltpu.sync_copy(idx_hbm.at[pl.ds(g, GRPS_PER_CHUNK)], idx_v)
        copies = [
            pltpu.async_copy(
                table_hbm.at[idx_v.at[j]],
                rows_v.at[pl.ds(j * GRP, GRP)],
                sem,
            )
            for j in range(GRPS_PER_CHUNK)
        ]
        for c in copies:
            c.wait()
        pltpu.sync_copy(rows_v, out_hbm.at[pl.ds(g * GRP, CHUNK_ROWS)])


@jax.jit
def kernel(X, tables):
    flat_table = tables.reshape(N_FIELDS * VOCAB, DIM)
    offsets = (jnp.arange(N_FIELDS, dtype=jnp.int32) * VOCAB)[None, :]
    flat_idx = (X.astype(jnp.int32) + offsets).reshape(N_GRPS, GRP)

    run = pl.kernel(
        _gather_kernel,
        out_type=jax.ShapeDtypeStruct((TOTAL_ROWS, DIM), jnp.float32),
        mesh=plsc.VectorSubcoreMesh(core_axis_name="c", subcore_axis_name="s"),
        scratch_types=[
            pltpu.VMEM((GRPS_PER_CHUNK, GRP), jnp.int32),
            pltpu.VMEM((CHUNK_ROWS, DIM), jnp.float32),
            pltpu.SemaphoreType.DMA,
        ],
        compiler_params=pltpu.CompilerParams(use_tc_tiling_on_sc=False),
    )
    out = run(flat_idx, flat_table)
    return out.reshape(BATCH, N_FIELDS, DIM)


# speedup vs baseline: 6.6032x; 6.6032x over previous
import jax
import jax.numpy as jnp
from jax import lax
from jax.experimental import pallas as pl
from jax.experimental.pallas import tpu as pltpu
from jax.experimental.pallas import tpu_sc as plsc

N_FIELDS = 26
VOCAB = 100000
DIM = 16
BATCH = 16384

NC = 2
FIELDS_PER_SC = N_FIELDS // NC
HALF_B = BATCH // 2


def _lookup_kernel(xt_hbm, tab_hbm, out_hbm, idx_v, slab_v, out_v, sem):
    c = lax.axis_index("c")
    s = lax.axis_index("s")

    @pl.loop(0, FIELDS_PER_SC)
    def _field(j):
        f = c * FIELDS_PER_SC + j
        pltpu.sync_copy(xt_hbm.at[f], idx_v)
        pltpu.sync_copy(tab_hbm.at[f, s], slab_v)

        @pl.loop(0, 2)
        def _half(h):
            @pl.loop(0, HALF_B // 16)
            def _grp(g):
                iv = idx_v[pl.ds(h * HALF_B + g * 16, 16)]
                out_v[pl.ds(g * 16, 16)] = plsc.load_gather(slab_v, [iv])

            pltpu.sync_copy(out_v,
                            out_hbm.at[f, s, pl.ds(h * HALF_B, HALF_B)])


@jax.jit
def kernel(X, tables):
    tab_t = jnp.transpose(tables, (0, 2, 1))
    xt = jnp.transpose(X.astype(jnp.int32))

    run = pl.kernel(
        _lookup_kernel,
        out_type=jax.ShapeDtypeStruct((N_FIELDS, DIM, BATCH), jnp.float32),
        mesh=plsc.VectorSubcoreMesh(core_axis_name="c", subcore_axis_name="s"),
        scratch_types=[
            pltpu.VMEM((BATCH,), jnp.int32),
            pltpu.VMEM((VOCAB,), jnp.float32),
            pltpu.VMEM((HALF_B,), jnp.float32),
            pltpu.SemaphoreType.DMA,
        ],
        compiler_params=pltpu.CompilerParams(needs_layout_passes=False),
    )
    out_t = run(xt, tab_t)
    return jnp.transpose(out_t, (2, 0, 1))


# speedup vs baseline: 6.9425x; 1.0514x over previous
import jax
import jax.numpy as jnp
from jax import lax
from jax.experimental import pallas as pl
from jax.experimental.pallas import tpu as pltpu
from jax.experimental.pallas import tpu_sc as plsc

N_FIELDS = 26
VOCAB = 100000
DIM = 16
BATCH = 16384

NC = 2
FIELDS_PER_SC = N_FIELDS // NC
HALF_B = BATCH // 2


QB = BATCH // 4


def _lookup_kernel(xt_hbm, tab_hbm, out_hbm, idx_v, slab_v,
                   out_a, out_b, sem_i, sem_s, sem_oa, sem_ob):
    c = lax.axis_index("c")
    s = lax.axis_index("s")

    @pl.loop(0, FIELDS_PER_SC)
    def _field(j):
        f = c * FIELDS_PER_SC + j
        ci = pltpu.async_copy(xt_hbm.at[f], idx_v, sem_i)
        cs = pltpu.async_copy(tab_hbm.at[f, s], slab_v, sem_s)
        ci.wait()
        cs.wait()

        def _quarter(q, out_v, sem_o):
            @pl.loop(0, QB // 16, unroll=8)
            def _grp(g):
                iv = idx_v[pl.ds(q * QB + g * 16, 16)]
                out_v[pl.ds(g * 16, 16)] = plsc.load_gather(slab_v, [iv])

            return pltpu.async_copy(
                out_v, out_hbm.at[f, s, pl.ds(q * QB, QB)], sem_o)

        w0 = _quarter(0, out_a, sem_oa)
        w1 = _quarter(1, out_b, sem_ob)
        w0.wait()
        w2 = _quarter(2, out_a, sem_oa)
        w1.wait()
        w3 = _quarter(3, out_b, sem_ob)
        w2.wait()
        w3.wait()


@jax.jit
def kernel(X, tables):
    tab_t = jnp.transpose(tables, (0, 2, 1))
    xt = jnp.transpose(X.astype(jnp.int32))

    run = pl.kernel(
        _lookup_kernel,
        out_type=jax.ShapeDtypeStruct((N_FIELDS, DIM, BATCH), jnp.float32),
        mesh=plsc.VectorSubcoreMesh(core_axis_name="c", subcore_axis_name="s"),
        scratch_types=[
            pltpu.VMEM((BATCH,), jnp.int32),
            pltpu.VMEM((VOCAB,), jnp.float32),
            pltpu.VMEM((QB,), jnp.float32),
            pltpu.VMEM((QB,), jnp.float32),
            pltpu.SemaphoreType.DMA,
            pltpu.SemaphoreType.DMA,
            pltpu.SemaphoreType.DMA,
            pltpu.SemaphoreType.DMA,
        ],
        compiler_params=pltpu.CompilerParams(needs_layout_passes=False),
    )
    out_t = run(xt, tab_t)
    return jnp.transpose(out_t, (2, 0, 1))


# speedup vs baseline: 12.3845x; 1.7839x over previous
import jax
import jax.numpy as jnp
from jax import lax
from jax.experimental import pallas as pl
from jax.experimental.pallas import tpu as pltpu
from jax.experimental.pallas import tpu_sc as plsc

N_FIELDS = 26
VOCAB = 100000
DIM = 16
BATCH = 16384

NC = 2
FIELDS_PER_SC = N_FIELDS // NC
HALF_B = BATCH // 2


QB = BATCH // 4


def _lookup_kernel(xt_hbm, tab_hbm, out_hbm, idx_v, slab_v,
                   out_a, out_b, sem_i, sem_s, sem_oa, sem_ob):
    c = lax.axis_index("c")
    s = lax.axis_index("s")

    @pl.loop(0, FIELDS_PER_SC)
    def _field(j):
        f = c * FIELDS_PER_SC + j
        ci = pltpu.async_copy(xt_hbm.at[f], idx_v, sem_i)
        cs = pltpu.async_copy(tab_hbm.at[f, s], slab_v, sem_s)
        ci.wait()
        cs.wait()

        def _quarter(q, out_v, sem_o):
            @pl.loop(0, QB // 128)
            def _sg(t):
                ivs = [idx_v[pl.ds(q * QB + t * 128 + i * 16, 16)]
                       for i in range(8)]
                vs = [plsc.load_gather(slab_v, [iv]) for iv in ivs]
                for i in range(8):
                    out_v[pl.ds(t * 128 + i * 16, 16)] = vs[i]

            return pltpu.async_copy(
                out_v, out_hbm.at[f, s, pl.ds(q * QB, QB)], sem_o)

        w0 = _quarter(0, out_a, sem_oa)
        w1 = _quarter(1, out_b, sem_ob)
        w0.wait()
        w2 = _quarter(2, out_a, sem_oa)
        w1.wait()
        w3 = _quarter(3, out_b, sem_ob)
        w2.wait()
        w3.wait()


@jax.jit
def kernel(X, tables):
    tab_t = jnp.transpose(tables, (0, 2, 1))
    xt = jnp.transpose(X.astype(jnp.int32))

    run = pl.kernel(
        _lookup_kernel,
        out_type=jax.ShapeDtypeStruct((N_FIELDS, DIM, BATCH), jnp.float32),
        mesh=plsc.VectorSubcoreMesh(core_axis_name="c", subcore_axis_name="s"),
        scratch_types=[
            pltpu.VMEM((BATCH,), jnp.int32),
            pltpu.VMEM((VOCAB,), jnp.float32),
            pltpu.VMEM((QB,), jnp.float32),
            pltpu.VMEM((QB,), jnp.float32),
            pltpu.SemaphoreType.DMA,
            pltpu.SemaphoreType.DMA,
            pltpu.SemaphoreType.DMA,
            pltpu.SemaphoreType.DMA,
        ],
        compiler_params=pltpu.CompilerParams(needs_layout_passes=False),
    )
    out_t = run(xt, tab_t)
    return jnp.transpose(out_t, (2, 0, 1))


# speedup vs baseline: 12.4471x; 1.0051x over previous
import jax
import jax.numpy as jnp
from jax import lax
from jax.experimental import pallas as pl
from jax.experimental.pallas import tpu as pltpu
from jax.experimental.pallas import tpu_sc as plsc

N_FIELDS = 26
VOCAB = 100000
DIM = 16
BATCH = 16384

NC = 2
FIELDS_PER_SC = N_FIELDS // NC
NQ = 4
QB = BATCH // NQ
N_STEPS = FIELDS_PER_SC * NQ


def _lookup_kernel(xt_hbm, tab_hbm, out_hbm,
                   idx_a, idx_b, slab_v, out_a, out_b,
                   sem_ia, sem_ib, sem_s, sem_oa, sem_ob):
    c = lax.axis_index("c")
    s = lax.axis_index("s")
    idxs = (idx_a, idx_b)
    sem_i = (sem_ia, sem_ib)
    outs = (out_a, out_b)
    sem_o = (sem_oa, sem_ob)

    def stage_idx(n):
        j, q = divmod(n, NQ)
        return pltpu.async_copy(
            xt_hbm.at[c * FIELDS_PER_SC + j, pl.ds(q * QB, QB)],
            idxs[n % 2], sem_i[n % 2])

    idx_pend = {0: stage_idx(0)}
    wb_pend = {}

    for j in range(FIELDS_PER_SC):
        f = c * FIELDS_PER_SC + j
        slab_pend = pltpu.async_copy(tab_hbm.at[f, s], slab_v, sem_s)

        for q in range(NQ):
            n = j * NQ + q
            if n + 1 < N_STEPS:
                idx_pend[n + 1] = stage_idx(n + 1)
            idx_pend.pop(n).wait()
            if q == 0:
                slab_pend.wait()
            if n % 2 in wb_pend:
                wb_pend.pop(n % 2).wait()
            idx_v = idxs[n % 2]
            out_v = outs[n % 2]

            @pl.loop(0, QB // 128)
            def _sg(t):
                ivs = [idx_v[pl.ds(t * 128 + i * 16, 16)]
                       for i in range(8)]
                vs = [plsc.load_gather(slab_v, [iv]) for iv in ivs]
                for i in range(8):
                    out_v[pl.ds(t * 128 + i * 16, 16)] = vs[i]

            wb_pend[n % 2] = pltpu.async_copy(
                out_v, out_hbm.at[f, s, pl.ds(q * QB, QB)], sem_o[n % 2])

    for w in wb_pend.values():
        w.wait()


@jax.jit
def kernel(X, tables):
    tab_t = jnp.transpose(tables, (0, 2, 1))
    xt = jnp.transpose(X.astype(jnp.int32))

    run = pl.kernel(
        _lookup_kernel,
        out_type=jax.ShapeDtypeStruct((N_FIELDS, DIM, BATCH), jnp.float32),
        mesh=plsc.VectorSubcoreMesh(core_axis_name="c", subcore_axis_name="s"),
        scratch_types=[
            pltpu.VMEM((QB,), jnp.int32),
            pltpu.VMEM((QB,), jnp.int32),
            pltpu.VMEM((VOCAB,), jnp.float32),
            pltpu.VMEM((QB,), jnp.float32),
            pltpu.VMEM((QB,), jnp.float32),
            pltpu.SemaphoreType.DMA,
            pltpu.SemaphoreType.DMA,
            pltpu.SemaphoreType.DMA,
            pltpu.SemaphoreType.DMA,
            pltpu.SemaphoreType.DMA,
        ],
        compiler_params=pltpu.CompilerParams(needs_layout_passes=False),
    )
    out_t = run(xt, tab_t)
    return jnp.transpose(out_t, (2, 0, 1))
